# baseline (speedup 1.0000x reference)
import functools

import jax
import jax.numpy as jnp
import numpy as np
from jax import lax
from jax.experimental import pallas as pl
from jax.experimental.pallas import tpu as pltpu
from jax.experimental.pallas import tpu_sc as plsc

N = 10000
E = 320000
IN_DIM = 128
HID = 64
HEADS = 4
NCLS = 64

NC = 2
NS = 16
NW = NC * NS
LANES = 16

EW = E // NW
BE = 80
NB = EW // BE
AW = 80
ACC_N = 10240
RPT = ACC_N // NS
ZR = 64

_HIGH = lax.Precision.DEFAULT

_S = np.array([(j // 32) * 32 + ((j % 32) * 2 if j % 32 < 16 else
                                 (j % 32 - 16) * 2 + 1) for j in range(64)])
_QP = np.argsort(_S)

_DNUMS = lax.GatherDimensionNumbers(
    offset_dims=(), collapsed_slice_dims=(0,), start_index_map=(0,))


def _bcast(v, t):
  idx = jnp.full((LANES, 1), t, jnp.int32)
  return lax.gather(v, idx, _DNUMS, slice_sizes=(1,),
                    mode=lax.GatherScatterMode.PROMISE_IN_BOUNDS)


def _make_sc_pass(num_heads, fdim):
  mesh = plsc.VectorSubcoreMesh(core_axis_name="c", subcore_axis_name="s")

  def body(src_hbm, dst_hbm, a_hbm, *rest):
    ft_hbms = rest[:num_heads]
    out_hbm = rest[num_heads]
    (srcv, dstv, a1v, a2v, rows0, rows1, rows2, stag0, stag1, stag2, zb,
     sg0, sg1, sg2, ss0, ss1, ss2, acc) = rest[num_heads + 1:]

    cid = lax.axis_index("c")
    sid = lax.axis_index("s")
    wid = sid * NC + cid

    pltpu.sync_copy(src_hbm.at[wid], srcv)
    pltpu.sync_copy(dst_hbm.at[wid], dstv)

    zeros16 = jnp.zeros((LANES,), jnp.float32)

    @pl.loop(0, ZR)
    def _(r):
      for q in range(AW // LANES):
        zb[r, pl.ds(q * LANES, LANES)] = zeros16

    row0 = sid * RPT
    for h in range(num_heads):
      for q in range(RPT // ZR):
        pltpu.sync_copy(zb, acc.at[pl.ds(row0 + q * ZR, ZR)])
      pltpu.sync_copy(a_hbm.at[h], a1v)
      pltpu.sync_copy(a_hbm.at[num_heads + h], a2v)
      plsc.subcore_barrier()

      ft_hbm = ft_hbms[h]

      def gather_start(j, rbuf, sem):
        pltpu.async_copy(ft_hbm.at[srcv.at[j, 0]], rbuf, sem)

      def gather_wait(rbuf, sem):
        pltpu.make_async_copy(ft_hbm.at[srcv.at[0, 0]], rbuf, sem).wait()

      def scatter_start(j, sbuf, sem):
        pltpu.async_copy(sbuf, acc.at[dstv.at[j, 0]], sem, add=True)

      def scatter_wait(sbuf, sem):
        pltpu.make_async_copy(sbuf, acc.at[dstv.at[0, 0]], sem).wait()

      def compute(j, rbuf, sbuf):
        @pl.loop(0, BE // LANES)
        def _(g):
          base = g * LANES
          s16 = srcv[j, 0, pl.ds(base, LANES)]
          d16 = dstv[j, 0, pl.ds(base, LANES)]
          e = plsc.load_gather(a1v, [d16]) + plsc.load_gather(a2v, [s16])
          e = jnp.where(e > 0, e, e * jnp.float32(0.01))
          p = jnp.exp(e)
          for t in range(LANES):
            pb = _bcast(p, t)
            er = base + t
            for q in range(fdim // 32):
              v = rbuf[er, pl.ds(q * 32, 32)]
              lo, hi = plsc.unpack(v, format=plsc.PackFormat.INTERLEAVED)
              sbuf[er, pl.ds(q * 32, LANES)] = lo * pb
              sbuf[er, pl.ds(q * 32 + LANES, LANES)] = hi * pb
            sbuf[er, pl.ds(fdim, LANES)] = pb

      for sbuf in (stag0, stag1, stag2):
        @pl.loop(0, BE)
        def _(r, sbuf=sbuf):
          for q in range(AW // LANES):
            sbuf[r, pl.ds(q * LANES, LANES)] = zeros16
      scatter_start(0, stag0, ss0)
      scatter_start(0, stag1, ss1)
      scatter_start(0, stag2, ss2)
      gather_start(0, rows0, sg0)
      gather_start(1, rows1, sg1)
      gather_start(2, rows2, sg2)

      bufs = ((rows0, stag0, sg0, ss0), (rows1, stag1, sg1, ss1),
              (rows2, stag2, sg2, ss2))

      @pl.loop(0, (NB - 2) // 3)
      def _(kk):
        jb = kk * 3
        for u, (rbuf, sbuf, sg, ss) in enumerate(bufs):
          j = jb + u
          scatter_wait(sbuf, ss)
          gather_wait(rbuf, sg)
          compute(j, rbuf, sbuf)
          jn = jnp.where(j + 3 >= NB, 0, j + 3)
          gather_start(jn, rbuf, sg)
          scatter_start(j, sbuf, ss)

      for u, (rbuf, sbuf, sg, ss) in enumerate(bufs[:2]):
        j = NB - 2 + u
        scatter_wait(sbuf, ss)
        gather_wait(rbuf, sg)
        compute(j, rbuf, sbuf)
        scatter_start(j, sbuf, ss)
      gather_wait(rows2, sg2)
      scatter_wait(stag0, ss0)
      scatter_wait(stag1, ss1)
      scatter_wait(stag2, ss2)

      plsc.subcore_barrier()
      pltpu.sync_copy(acc.at[pl.ds(row0, RPT)],
                      out_hbm.at[h, cid, pl.ds(row0, RPT)])

  scratch = [
      pltpu.VMEM((NB, 1, BE), jnp.int32),
      pltpu.VMEM((NB, 1, BE), jnp.int32),
      pltpu.VMEM((N,), jnp.float32),
      pltpu.VMEM((N,), jnp.float32),
      pltpu.VMEM((BE, fdim), jnp.bfloat16),
      pltpu.VMEM((BE, fdim), jnp.bfloat16),
      pltpu.VMEM((BE, fdim), jnp.bfloat16),
      pltpu.VMEM((BE, AW), jnp.float32),
      pltpu.VMEM((BE, AW), jnp.float32),
      pltpu.VMEM((BE, AW), jnp.float32),
      pltpu.VMEM((ZR, AW), jnp.float32),
      pltpu.SemaphoreType.DMA,
      pltpu.SemaphoreType.DMA,
      pltpu.SemaphoreType.DMA,
      pltpu.SemaphoreType.DMA,
      pltpu.SemaphoreType.DMA,
      pltpu.SemaphoreType.DMA,
      pltpu.VMEM_SHARED((ACC_N, AW), jnp.float32),
  ]
  return pl.kernel(
      body,
      out_type=jax.ShapeDtypeStruct((num_heads, NC, ACC_N, AW), jnp.float32),
      mesh=mesh,
      scratch_types=scratch,
      compiler_params=pltpu.CompilerParams(
          needs_layout_passes=False, use_tc_tiling_on_sc=False),
  )


def _tc1_body(x_ref, w_ref, b_ref, alr_ref, ab_ref, ft_ref, a_ref):
  x = x_ref[...]
  ft = jnp.dot(x, w_ref[...], precision=_HIGH) + b_ref[...]
  a = jnp.dot(ft, alr_ref[...], precision=_HIGH) + ab_ref[...]
  a_ref[...] = a
  r = ft.shape[0]
  ft_ref[...] = jnp.transpose(ft.reshape(r, HEADS, HID),
                              (1, 0, 2)).astype(jnp.bfloat16)


def _tc2_body(p_ref, wf_ref, bf_ref, afw_ref, afb_ref, ftf_ref, af_ref):
  heads = []
  for h in range(HEADS):
    ph = p_ref[h, 0] + p_ref[h, 1]
    numer = ph[:, 0:HID]
    den = jnp.maximum(ph[:, HID:HID + 1], jnp.float32(1e-16))
    heads.append(jnp.maximum(numer / den, 0.0))
  last = jnp.concatenate(heads, axis=1)
  ftf = jnp.dot(last, wf_ref[...], precision=_HIGH) + bf_ref[...]
  ftf_ref[...] = ftf.astype(jnp.bfloat16)
  af = jnp.dot(ftf, afw_ref[...], precision=_HIGH) + afb_ref[...]
  af_ref[...] = af


def _tc3_body(p_ref, o_ref):
  ph = p_ref[0] + p_ref[1]
  den = jnp.maximum(ph[:, NCLS:NCLS + 1], jnp.float32(1e-16))
  o_ref[...] = jnp.maximum(ph[:, 0:NCLS] / den, 0.0)


_R = 2000
_G = N // _R


def _full(shape):
  return pl.BlockSpec(shape, lambda i: tuple(0 for _ in shape))


_tc1 = pl.pallas_call(
    _tc1_body,
    grid=(_G,),
    in_specs=[
        pl.BlockSpec((_R, IN_DIM), lambda i: (i, 0)),
        _full((IN_DIM, HEADS * HID)),
        _full((HEADS * HID,)),
        _full((HEADS * HID, 2 * HEADS)),
        _full((2 * HEADS,)),
    ],
    out_specs=[
        pl.BlockSpec((HEADS, _R, HID), lambda i: (0, i, 0)),
        pl.BlockSpec((_R, 2 * HEADS), lambda i: (i, 0)),
    ],
    out_shape=[
        jax.ShapeDtypeStruct((HEADS, N, HID), jnp.bfloat16),
        jax.ShapeDtypeStruct((N, 2 * HEADS), jnp.float32),
    ],
)

_tc2 = pl.pallas_call(
    _tc2_body,
    grid=(_G,),
    in_specs=[
        pl.BlockSpec((HEADS, NC, _R, AW), lambda i: (0, 0, i, 0)),

        _full((HEADS * HID, NCLS)),
        _full((NCLS,)),
        _full((NCLS, 2)),
        _full((2,)),
    ],
    out_specs=[
        pl.BlockSpec((_R, NCLS), lambda i: (i, 0)),
        pl.BlockSpec((_R, 2), lambda i: (i, 0)),
    ],
    out_shape=[
        jax.ShapeDtypeStruct((N, NCLS), jnp.bfloat16),
        jax.ShapeDtypeStruct((N, 2), jnp.float32),
    ],
)

_tc3 = pl.pallas_call(
    _tc3_body,
    grid=(_G,),
    in_specs=[pl.BlockSpec((NC, _R, AW), lambda i: (0, i, 0))],
    out_specs=pl.BlockSpec((_R, NCLS), lambda i: (i, 0)),
    out_shape=jax.ShapeDtypeStruct((N, NCLS), jnp.float32),
)

_sc_layer0 = _make_sc_pass(HEADS, HID)
_sc_final = _make_sc_pass(1, NCLS)


def kernel(features, edge_index, W1, b1, al_w, al_b, ar_w, ar_b,
           Wf, bf, alf_w, alf_b, arf_w, arf_b):
  qp = jnp.asarray(_QP)
  w1c = jnp.transpose(W1[:, :, _QP], (1, 0, 2)).reshape(IN_DIM, HEADS * HID)
  b1c = b1[:, qp].reshape(HEADS * HID)
  eye = jnp.eye(HEADS, dtype=jnp.float32)
  alr = jnp.concatenate(
      [(al_w[:, qp, None] * eye[:, None, :]).reshape(HEADS * HID, HEADS),
       (ar_w[:, qp, None] * eye[:, None, :]).reshape(HEADS * HID, HEADS)],
      axis=1)
  ab = jnp.concatenate([al_b, ar_b])
  src = edge_index[0].reshape(NW, NB, 1, BE)
  dst = edge_index[1].reshape(NW, NB, 1, BE)

  ft_heads, a_nt = _tc1(features, w1c, b1c, alr, ab)
  a_t = a_nt.T
  fts = [ft_heads[h] for h in range(HEADS)]
  p0 = _sc_layer0(src, dst, a_t, *fts)

  afw = jnp.stack([alf_w[qp], arf_w[qp]], axis=1)
  afb = jnp.stack([alf_b, arf_b])
  ftf, af_nt = _tc2(p0, Wf[:, qp], bf[qp], afw, afb)
  pf = _sc_final(src, dst, af_nt.T, ftf)
  return _tc3(pf[0])


# speedup vs baseline: 1.0081x; 1.0081x over previous
import jax
import jax.numpy as jnp
import numpy as np
from jax import lax
from jax.experimental import pallas as pl
from jax.experimental.pallas import tpu as pltpu
from jax.experimental.pallas import tpu_sc as plsc

N = 10000
E = 320000
IN_DIM = 128
HID = 64
HEADS = 4
NCLS = 64

NC = 2
NS = 16
NW = NC * NS
LANES = 16

EW = E // NW
BE = 80
NB = EW // BE
AW = 80
ACC_N = 10240
RPT = ACC_N // NS
ZR = 64

_HIGH = lax.Precision.DEFAULT

_S = np.array([(j // 32) * 32 + ((j % 32) * 2 if j % 32 < 16 else
                                 (j % 32 - 16) * 2 + 1) for j in range(64)])
_QP = np.argsort(_S)

_DNUMS = lax.GatherDimensionNumbers(
    offset_dims=(), collapsed_slice_dims=(0,), start_index_map=(0,))


def _bcast(v, t):
  idx = jnp.full((LANES, 1), t, jnp.int32)
  return lax.gather(v, idx, _DNUMS, slice_sizes=(1,),
                    mode=lax.GatherScatterMode.PROMISE_IN_BOUNDS)


def _make_sc_pass(num_heads, fdim):
  mesh = plsc.VectorSubcoreMesh(core_axis_name="c", subcore_axis_name="s")

  def body(src_hbm, dst_hbm, a_hbm, *rest):
    ft_hbms = rest[:num_heads]
    out_hbm = rest[num_heads]
    (srcv, dstv, a1v, a2v, rows0, rows1, stag0, stag1, zb, padv,
     sg0, sg1, ss0, ss1, acc) = rest[num_heads + 1:]

    cid = lax.axis_index("c")
    sid = lax.axis_index("s")
    wid = sid * NC + cid

    pltpu.sync_copy(src_hbm.at[wid], srcv)
    pltpu.sync_copy(dst_hbm.at[wid], dstv)

    zeros16 = jnp.zeros((LANES,), jnp.float32)

    @pl.loop(0, ZR)
    def _(r):
      for q in range(AW // LANES):
        zb[r, pl.ds(q * LANES, LANES)] = zeros16

    for g in range(BE // LANES):
      padv[0, 0, pl.ds(g * LANES, LANES)] = (
          lax.iota(jnp.int32, LANES) + (N + g * LANES))

    row0 = sid * RPT
    for h in range(num_heads):
      for q in range(RPT // ZR):
        pltpu.sync_copy(zb, acc.at[pl.ds(row0 + q * ZR, ZR)])
      pltpu.sync_copy(a_hbm.at[h], a1v)
      pltpu.sync_copy(a_hbm.at[num_heads + h], a2v)
      plsc.subcore_barrier()

      ft_hbm = ft_hbms[h]

      def gather_start(j, rbuf, sem):
        pltpu.async_copy(ft_hbm.at[srcv.at[j, 0]], rbuf, sem)

      def gather_wait(rbuf, sem):
        pltpu.make_async_copy(ft_hbm.at[srcv.at[0, 0]], rbuf, sem).wait()

      def scatter_start(j, sbuf, sem):
        pltpu.async_copy(sbuf, acc.at[dstv.at[j, 0]], sem, add=True)

      def scatter_wait(sbuf, sem):
        pltpu.make_async_copy(sbuf, acc.at[dstv.at[0, 0]], sem).wait()

      def compute(j, rbuf, sbuf):
        @pl.loop(0, BE // LANES)
        def _(g):
          base = g * LANES
          s16 = srcv[j, 0, pl.ds(base, LANES)]
          d16 = dstv[j, 0, pl.ds(base, LANES)]
          e = plsc.load_gather(a1v, [d16]) + plsc.load_gather(a2v, [s16])
          e = jnp.where(e > 0, e, e * jnp.float32(0.01))
          p = jnp.exp(e)
          for t in range(LANES):
            pb = _bcast(p, t)
            er = base + t
            for q in range(fdim // 32):
              v = rbuf[er, pl.ds(q * 32, 32)]
              lo, hi = plsc.unpack(v, format=plsc.PackFormat.INTERLEAVED)
              sbuf[er, pl.ds(q * 32, LANES)] = lo * pb
              sbuf[er, pl.ds(q * 32 + LANES, LANES)] = hi * pb
            sbuf[er, pl.ds(fdim, LANES)] = pb

      pltpu.async_copy(stag0, acc.at[padv.at[0, 0]], ss0, add=True)
      pltpu.async_copy(stag1, acc.at[padv.at[0, 0]], ss1, add=True)
      gather_start(0, rows0, sg0)
      gather_start(1, rows1, sg1)

      @pl.loop(0, (NB - 1) // 2)
      def _(jj):
        j0 = jj * 2
        j1 = j0 + 1
        scatter_wait(stag0, ss0)
        gather_wait(rows0, sg0)
        compute(j0, rows0, stag0)
        gather_start(j0 + 2, rows0, sg0)
        scatter_start(j0, stag0, ss0)
        scatter_wait(stag1, ss1)
        gather_wait(rows1, sg1)
        compute(j1, rows1, stag1)
        jn = jnp.where(j1 + 2 >= NB, 0, j1 + 2)
        gather_start(jn, rows1, sg1)
        scatter_start(j1, stag1, ss1)

      scatter_wait(stag0, ss0)
      gather_wait(rows0, sg0)
      compute(NB - 1, rows0, stag0)
      scatter_start(NB - 1, stag0, ss0)
      scatter_wait(stag0, ss0)
      gather_wait(rows1, sg1)
      scatter_wait(stag1, ss1)

      plsc.subcore_barrier()
      pltpu.sync_copy(acc.at[pl.ds(row0, RPT)],
                      out_hbm.at[h, cid, pl.ds(row0, RPT)])

  scratch = [
      pltpu.VMEM((NB, 1, BE), jnp.int32),
      pltpu.VMEM((NB, 1, BE), jnp.int32),
      pltpu.VMEM((N,), jnp.float32),
      pltpu.VMEM((N,), jnp.float32),
      pltpu.VMEM((BE, fdim), jnp.bfloat16),
      pltpu.VMEM((BE, fdim), jnp.bfloat16),
      pltpu.VMEM((BE, AW), jnp.float32),
      pltpu.VMEM((BE, AW), jnp.float32),
      pltpu.VMEM((ZR, AW), jnp.float32),
      pltpu.VMEM((1, 1, BE), jnp.int32),
      pltpu.SemaphoreType.DMA,
      pltpu.SemaphoreType.DMA,
      pltpu.SemaphoreType.DMA,
      pltpu.SemaphoreType.DMA,
      pltpu.VMEM_SHARED((ACC_N, AW), jnp.float32),
  ]
  return pl.kernel(
      body,
      out_type=jax.ShapeDtypeStruct((num_heads, NC, ACC_N, AW), jnp.float32),
      mesh=mesh,
      scratch_types=scratch,
      compiler_params=pltpu.CompilerParams(
          needs_layout_passes=False, use_tc_tiling_on_sc=False),
  )


def _tc1_body(x_ref, w_ref, b_ref, alr_ref, ab_ref, ft_ref, a_ref):
  x = x_ref[...]
  ft = jnp.dot(x, w_ref[...], precision=_HIGH) + b_ref[...]
  a = jnp.dot(ft, alr_ref[...], precision=_HIGH) + ab_ref[...]
  a_ref[...] = a
  r = ft.shape[0]
  ft_ref[...] = jnp.transpose(ft.reshape(r, HEADS, HID),
                              (1, 0, 2)).astype(jnp.bfloat16)


def _tc2_body(p_ref, wf_ref, bf_ref, afw_ref, afb_ref, ftf_ref, af_ref):
  heads = []
  for h in range(HEADS):
    ph = p_ref[h, 0] + p_ref[h, 1]
    numer = ph[:, 0:HID]
    den = jnp.maximum(ph[:, HID:HID + 1], jnp.float32(1e-16))
    heads.append(jnp.maximum(numer / den, 0.0))
  last = jnp.concatenate(heads, axis=1)
  ftf = jnp.dot(last, wf_ref[...], precision=_HIGH) + bf_ref[...]
  ftf_ref[...] = ftf.astype(jnp.bfloat16)
  af = jnp.dot(ftf, afw_ref[...], precision=_HIGH) + afb_ref[...]
  af_ref[...] = af


def _tc3_body(p_ref, o_ref):
  ph = p_ref[0] + p_ref[1]
  den = jnp.maximum(ph[:, NCLS:NCLS + 1], jnp.float32(1e-16))
  o_ref[...] = jnp.maximum(ph[:, 0:NCLS] / den, 0.0)


_R = 2000
_G = N // _R


def _full(shape):
  return pl.BlockSpec(shape, lambda i: tuple(0 for _ in shape))


_tc1 = pl.pallas_call(
    _tc1_body,
    grid=(_G,),
    in_specs=[
        pl.BlockSpec((_R, IN_DIM), lambda i: (i, 0)),
        _full((IN_DIM, HEADS * HID)),
        _full((HEADS * HID,)),
        _full((HEADS * HID, 2 * HEADS)),
        _full((2 * HEADS,)),
    ],
    out_specs=[
        pl.BlockSpec((HEADS, _R, HID), lambda i: (0, i, 0)),
        pl.BlockSpec((_R, 2 * HEADS), lambda i: (i, 0)),
    ],
    out_shape=[
        jax.ShapeDtypeStruct((HEADS, N, HID), jnp.bfloat16),
        jax.ShapeDtypeStruct((N, 2 * HEADS), jnp.float32),
    ],
)

_tc2 = pl.pallas_call(
    _tc2_body,
    grid=(_G,),
    in_specs=[
        pl.BlockSpec((HEADS, NC, _R, AW), lambda i: (0, 0, i, 0)),

        _full((HEADS * HID, NCLS)),
        _full((NCLS,)),
        _full((NCLS, 2)),
        _full((2,)),
    ],
    out_specs=[
        pl.BlockSpec((_R, NCLS), lambda i: (i, 0)),
        pl.BlockSpec((_R, 2), lambda i: (i, 0)),
    ],
    out_shape=[
        jax.ShapeDtypeStruct((N, NCLS), jnp.bfloat16),
        jax.ShapeDtypeStruct((N, 2), jnp.float32),
    ],
)

_tc3 = pl.pallas_call(
    _tc3_body,
    grid=(_G,),
    in_specs=[pl.BlockSpec((NC, _R, AW), lambda i: (0, i, 0))],
    out_specs=pl.BlockSpec((_R, NCLS), lambda i: (i, 0)),
    out_shape=jax.ShapeDtypeStruct((N, NCLS), jnp.float32),
)

_sc_layer0 = _make_sc_pass(HEADS, HID)
_sc_final = _make_sc_pass(1, NCLS)


def kernel(features, edge_index, W1, b1, al_w, al_b, ar_w, ar_b,
           Wf, bf, alf_w, alf_b, arf_w, arf_b):
  qp = jnp.asarray(_QP)
  w1c = jnp.transpose(W1[:, :, _QP], (1, 0, 2)).reshape(IN_DIM, HEADS * HID)
  b1c = b1[:, qp].reshape(HEADS * HID)
  eye = jnp.eye(HEADS, dtype=jnp.float32)
  alr = jnp.concatenate(
      [(al_w[:, qp, None] * eye[:, None, :]).reshape(HEADS * HID, HEADS),
       (ar_w[:, qp, None] * eye[:, None, :]).reshape(HEADS * HID, HEADS)],
      axis=1)
  ab = jnp.concatenate([al_b, ar_b])
  src = edge_index[0].reshape(NW, NB, 1, BE)
  dst = edge_index[1].reshape(NW, NB, 1, BE)

  ft_heads, a_nt = _tc1(features, w1c, b1c, alr, ab)
  a_t = a_nt.T
  fts = [ft_heads[h] for h in range(HEADS)]
  p0 = _sc_layer0(src, dst, a_t, *fts)

  afw = jnp.stack([alf_w[qp], arf_w[qp]], axis=1)
  afb = jnp.stack([alf_b, arf_b])
  ftf, af_nt = _tc2(p0, Wf[:, qp], bf[qp], afw, afb)
  pf = _sc_final(src, dst, af_nt.T, ftf)
  return _tc3(pf[0])


# speedup vs baseline: 1.0097x; 1.0016x over previous
import jax
import jax.numpy as jnp
import numpy as np
from jax import lax
from jax.experimental import pallas as pl
from jax.experimental.pallas import tpu as pltpu
from jax.experimental.pallas import tpu_sc as plsc

N = 10000
E = 320000
IN_DIM = 128
HID = 64
HEADS = 4
NCLS = 64

NC = 2
NS = 16
NW = NC * NS
LANES = 16

EW = E // NW
BE = 80
NB = EW // BE
AW = 80
ACC_N = 10240
RPT = ACC_N // NS
ZR = 64

_HIGH = lax.Precision.DEFAULT

_S = np.array([(j // 32) * 32 + ((j % 32) * 2 if j % 32 < 16 else
                                 (j % 32 - 16) * 2 + 1) for j in range(64)])
_QP = np.argsort(_S)

_DNUMS = lax.GatherDimensionNumbers(
    offset_dims=(), collapsed_slice_dims=(0,), start_index_map=(0,))


def _bcast(v, t):
  idx = jnp.full((LANES, 1), t, jnp.int32)
  return lax.gather(v, idx, _DNUMS, slice_sizes=(1,),
                    mode=lax.GatherScatterMode.PROMISE_IN_BOUNDS)


def _make_sc_pass(num_heads, fdim):
  mesh = plsc.VectorSubcoreMesh(core_axis_name="c", subcore_axis_name="s",
                                num_cores=NC, num_subcores=NS)

  def body(src_hbm, dst_hbm, a_hbm, *rest):
    ft_hbms = rest[:num_heads]
    out_hbm = rest[num_heads]
    (srcv, dstv, a1v, a2v, rows0, rows1, stag0, stag1, zb, padv,
     sg0, sg1, ss0, ss1, acc) = rest[num_heads + 1:]

    cid = lax.axis_index("c")
    sid = lax.axis_index("s")
    wid = sid * NC + cid

    pltpu.sync_copy(src_hbm.at[wid], srcv)
    pltpu.sync_copy(dst_hbm.at[wid], dstv)

    zeros16 = jnp.zeros((LANES,), jnp.float32)

    @pl.loop(0, ZR)
    def _(r):
      for q in range(AW // LANES):
        zb[r, pl.ds(q * LANES, LANES)] = zeros16

    for g in range(BE // LANES):
      padv[0, 0, pl.ds(g * LANES, LANES)] = (
          lax.iota(jnp.int32, LANES) + (N + g * LANES))

    row0 = sid * RPT
    for h in range(num_heads):
      for q in range(RPT // ZR):
        pltpu.sync_copy(zb, acc.at[pl.ds(row0 + q * ZR, ZR)])
      pltpu.sync_copy(a_hbm.at[h], a1v)
      pltpu.sync_copy(a_hbm.at[num_heads + h], a2v)
      plsc.subcore_barrier()

      ft_hbm = ft_hbms[h]

      def gather_start(j, rbuf, sem):
        pltpu.async_copy(ft_hbm.at[srcv.at[j, 0]], rbuf, sem)

      def gather_wait(rbuf, sem):
        pltpu.make_async_copy(ft_hbm.at[srcv.at[0, 0]], rbuf, sem).wait()

      def scatter_start(j, sbuf, sem):
        pltpu.async_copy(sbuf, acc.at[dstv.at[j, 0]], sem, add=True)

      def scatter_wait(sbuf, sem):
        pltpu.make_async_copy(sbuf, acc.at[dstv.at[0, 0]], sem).wait()

      def compute(j, rbuf, sbuf):
        @pl.loop(0, BE // LANES)
        def _(g):
          base = g * LANES
          s16 = srcv[j, 0, pl.ds(base, LANES)]
          d16 = dstv[j, 0, pl.ds(base, LANES)]
          e = plsc.load_gather(a1v, [d16]) + plsc.load_gather(a2v, [s16])
          e = jnp.where(e > 0, e, e * jnp.float32(0.01))
          p = jnp.exp(e)
          for t in range(LANES):
            pb = _bcast(p, t)
            er = base + t
            for q in range(fdim // 32):
              v = rbuf[er, pl.ds(q * 32, 32)]
              lo, hi = plsc.unpack(v, format=plsc.PackFormat.INTERLEAVED)
              sbuf[er, pl.ds(q * 32, LANES)] = lo * pb
              sbuf[er, pl.ds(q * 32 + LANES, LANES)] = hi * pb
            sbuf[er, pl.ds(fdim, LANES)] = pb

      pltpu.async_copy(stag0, acc.at[padv.at[0, 0]], ss0, add=True)
      pltpu.async_copy(stag1, acc.at[padv.at[0, 0]], ss1, add=True)
      gather_start(0, rows0, sg0)
      gather_start(1, rows1, sg1)

      @pl.loop(0, (NB - 1) // 2)
      def _(jj):
        j0 = jj * 2
        j1 = j0 + 1
        scatter_wait(stag0, ss0)
        gather_wait(rows0, sg0)
        compute(j0, rows0, stag0)
        gather_start(j0 + 2, rows0, sg0)
        scatter_start(j0, stag0, ss0)
        scatter_wait(stag1, ss1)
        gather_wait(rows1, sg1)
        compute(j1, rows1, stag1)
        jn = jnp.where(j1 + 2 >= NB, 0, j1 + 2)
        gather_start(jn, rows1, sg1)
        scatter_start(j1, stag1, ss1)

      scatter_wait(stag0, ss0)
      gather_wait(rows0, sg0)
      compute(NB - 1, rows0, stag0)
      scatter_start(NB - 1, stag0, ss0)
      scatter_wait(stag0, ss0)
      gather_wait(rows1, sg1)
      scatter_wait(stag1, ss1)

      plsc.subcore_barrier()
      pltpu.sync_copy(acc.at[pl.ds(row0, RPT)],
                      out_hbm.at[h, cid, pl.ds(row0, RPT)])

  scratch = [
      pltpu.VMEM((NB, 1, BE), jnp.int32),
      pltpu.VMEM((NB, 1, BE), jnp.int32),
      pltpu.VMEM((N,), jnp.float32),
      pltpu.VMEM((N,), jnp.float32),
      pltpu.VMEM((BE, fdim), jnp.bfloat16),
      pltpu.VMEM((BE, fdim), jnp.bfloat16),
      pltpu.VMEM((BE, AW), jnp.float32),
      pltpu.VMEM((BE, AW), jnp.float32),
      pltpu.VMEM((ZR, AW), jnp.float32),
      pltpu.VMEM((1, 1, BE), jnp.int32),
      pltpu.SemaphoreType.DMA,
      pltpu.SemaphoreType.DMA,
      pltpu.SemaphoreType.DMA,
      pltpu.SemaphoreType.DMA,
      pltpu.VMEM_SHARED((ACC_N, AW), jnp.float32),
  ]
  return pl.kernel(
      body,
      out_type=jax.ShapeDtypeStruct((num_heads, NC, ACC_N, AW), jnp.float32),
      mesh=mesh,
      scratch_types=scratch,
      compiler_params=pltpu.CompilerParams(
          needs_layout_passes=False, use_tc_tiling_on_sc=False),
  )


def _tc1_body(x_ref, w_ref, b_ref, alr_ref, ab_ref, ft_ref, a_ref):
  x = x_ref[...]
  ft = jnp.dot(x, w_ref[...], precision=_HIGH) + b_ref[...]
  a = jnp.dot(ft, alr_ref[...], precision=_HIGH) + ab_ref[...]
  a_ref[...] = a
  r = ft.shape[0]
  ft_ref[...] = jnp.transpose(ft.reshape(r, HEADS, HID),
                              (1, 0, 2)).astype(jnp.bfloat16)


def _tc2_body(p_ref, wf_ref, bf_ref, afw_ref, afb_ref, ftf_ref, af_ref):
  heads = []
  for h in range(HEADS):
    ph = p_ref[h, 0] + p_ref[h, 1]
    numer = ph[:, 0:HID]
    den = jnp.maximum(ph[:, HID:HID + 1], jnp.float32(1e-16))
    heads.append(jnp.maximum(numer / den, 0.0))
  last = jnp.concatenate(heads, axis=1)
  ftf = jnp.dot(last, wf_ref[...], precision=_HIGH) + bf_ref[...]
  ftf_ref[...] = ftf.astype(jnp.bfloat16)
  af = jnp.dot(ftf, afw_ref[...], precision=_HIGH) + afb_ref[...]
  af_ref[...] = af


def _tc3_body(p_ref, o_ref):
  ph = p_ref[0] + p_ref[1]
  den = jnp.maximum(ph[:, NCLS:NCLS + 1], jnp.float32(1e-16))
  o_ref[...] = jnp.maximum(ph[:, 0:NCLS] / den, 0.0)


_R = 2000
_G = N // _R


def _full(shape):
  return pl.BlockSpec(shape, lambda i: tuple(0 for _ in shape))


_tc1 = pl.pallas_call(
    _tc1_body,
    grid=(_G,),
    in_specs=[
        pl.BlockSpec((_R, IN_DIM), lambda i: (i, 0)),
        _full((IN_DIM, HEADS * HID)),
        _full((HEADS * HID,)),
        _full((HEADS * HID, 2 * HEADS)),
        _full((2 * HEADS,)),
    ],
    out_specs=[
        pl.BlockSpec((HEADS, _R, HID), lambda i: (0, i, 0)),
        pl.BlockSpec((_R, 2 * HEADS), lambda i: (i, 0)),
    ],
    out_shape=[
        jax.ShapeDtypeStruct((HEADS, N, HID), jnp.bfloat16),
        jax.ShapeDtypeStruct((N, 2 * HEADS), jnp.float32),
    ],
)

_tc2 = pl.pallas_call(
    _tc2_body,
    grid=(_G,),
    in_specs=[
        pl.BlockSpec((HEADS, NC, _R, AW), lambda i: (0, 0, i, 0)),

        _full((HEADS * HID, NCLS)),
        _full((NCLS,)),
        _full((NCLS, 2)),
        _full((2,)),
    ],
    out_specs=[
        pl.BlockSpec((_R, NCLS), lambda i: (i, 0)),
        pl.BlockSpec((_R, 2), lambda i: (i, 0)),
    ],
    out_shape=[
        jax.ShapeDtypeStruct((N, NCLS), jnp.bfloat16),
        jax.ShapeDtypeStruct((N, 2), jnp.float32),
    ],
)

_tc3 = pl.pallas_call(
    _tc3_body,
    grid=(_G,),
    in_specs=[pl.BlockSpec((NC, _R, AW), lambda i: (0, i, 0))],
    out_specs=pl.BlockSpec((_R, NCLS), lambda i: (i, 0)),
    out_shape=jax.ShapeDtypeStruct((N, NCLS), jnp.float32),
)

_sc_layer0 = _make_sc_pass(HEADS, HID)
_sc_final = _make_sc_pass(1, NCLS)


def kernel(features, edge_index, W1, b1, al_w, al_b, ar_w, ar_b,
           Wf, bf, alf_w, alf_b, arf_w, arf_b):
  qp = jnp.asarray(_QP)
  w1c = jnp.transpose(W1[:, :, _QP], (1, 0, 2)).reshape(IN_DIM, HEADS * HID)
  b1c = b1[:, qp].reshape(HEADS * HID)
  eye = jnp.eye(HEADS, dtype=jnp.float32)
  alr = jnp.concatenate(
      [(al_w[:, qp, None] * eye[:, None, :]).reshape(HEADS * HID, HEADS),
       (ar_w[:, qp, None] * eye[:, None, :]).reshape(HEADS * HID, HEADS)],
      axis=1)
  ab = jnp.concatenate([al_b, ar_b])
  src = edge_index[0].reshape(NW, NB, 1, BE)
  dst = edge_index[1].reshape(NW, NB, 1, BE)

  ft_heads, a_nt = _tc1(features, w1c, b1c, alr, ab)
  a_t = a_nt.T
  fts = [ft_heads[h] for h in range(HEADS)]
  p0 = _sc_layer0(src, dst, a_t, *fts)

  afw = jnp.stack([alf_w[qp], arf_w[qp]], axis=1)
  afb = jnp.stack([alf_b, arf_b])
  ftf, af_nt = _tc2(p0, Wf[:, qp], bf[qp], afw, afb)
  pf = _sc_final(src, dst, af_nt.T, ftf)
  return _tc3(pf[0])
